# Initial kernel scaffold; baseline (speedup 1.0000x reference)
import jax
import jax.numpy as jnp
from jax.experimental import pallas as pl


def kernel(entity_emb, heads, rels, tails, rel_table, W):
    raise NotImplementedError("write your pallas kernel here")


# speedup vs baseline: 2.1609x; 2.1609x over previous
import functools

import jax
import jax.numpy as jnp
from jax import lax
from jax.experimental import pallas as pl
from jax.experimental.pallas import tpu as pltpu
from jax.experimental.pallas import tpu_sc as plsc

N_ENT = 10000
N_REL = 64
N_EDGE = 160000
D = 256
DH = 128
NC = 2
NS = 16
L = 16
EW = N_EDGE // NS
C = 80
NCHUNK = EW // C
RPA = 624
RTL = N_ENT - NS * RPA

_mesh = plsc.VectorSubcoreMesh(core_axis_name="c", subcore_axis_name="s")
_sc_params = pltpu.CompilerParams(needs_layout_passes=False)


def _tanh_via_exp(x):
    ex = jnp.exp(x + x)
    return 1.0 - 2.0 / (ex + 1.0)


@functools.partial(
    pl.kernel,
    out_type=jax.ShapeDtypeStruct((NC * N_EDGE,), jnp.float32),
    mesh=_mesh,
    scratch_types=[
        pltpu.VMEM((C,), jnp.int32),
        pltpu.VMEM((C,), jnp.int32),
        pltpu.VMEM((C,), jnp.int32),
        pltpu.VMEM((C, DH), jnp.float32),
        pltpu.VMEM((C, DH), jnp.float32),
        pltpu.VMEM((C, DH), jnp.float32),
        pltpu.VMEM((C,), jnp.float32),
        pltpu.SemaphoreType.DMA,
    ],
    compiler_params=_sc_params,
)
def _phase_a(ent_cat, rel_cat, h_idx, t_idx, r_idx, scores_out,
             hv, tv, rv, ehb, etb, erb, sbuf, sem):
    c = lax.axis_index("c")
    s = lax.axis_index("s")

    def chunk(i, _):
        eb = c * N_EDGE + s * EW + i * C
        pltpu.sync_copy(h_idx.at[pl.ds(eb, C)], hv)
        pltpu.sync_copy(t_idx.at[pl.ds(eb, C)], tv)
        pltpu.sync_copy(r_idx.at[pl.ds(eb, C)], rv)
        d1 = pltpu.async_copy(ent_cat.at[hv], ehb, sem)
        d2 = pltpu.async_copy(ent_cat.at[tv], etb, sem)
        d3 = pltpu.async_copy(rel_cat.at[rv], erb, sem)
        d1.wait()
        d2.wait()
        d3.wait()

        z16 = jnp.zeros((L,), jnp.float32)
        for j in range(C // L):
            sbuf[pl.ds(j * L, L)] = z16
        zi16 = jnp.zeros((L,), jnp.int32)

        def edge(e, _):
            acc = jnp.zeros((L,), jnp.float32)
            for k in range(DH // L):
                vh = ehb[e, pl.ds(k * L, L)]
                vr = erb[e, pl.ds(k * L, L)]
                vt = etb[e, pl.ds(k * L, L)]
                acc = acc + vt * _tanh_via_exp(vh + vr)
            plsc.addupdate_scatter(sbuf, [zi16 + e], acc)
            return 0

        lax.fori_loop(0, C, edge, 0)
        pltpu.sync_copy(sbuf, scores_out.at[pl.ds(eb, C)])
        return 0

    lax.fori_loop(0, NCHUNK, chunk, 0)


def _mid_body(sp_ref, p_ref):
    sc = sp_ref[0, :] + sp_ref[1, :]
    m = jnp.max(sc)
    p_ref[...] = jnp.exp(sc - m)[None, :]


@functools.partial(
    pl.kernel,
    out_type=[
        jax.ShapeDtypeStruct((NC * N_ENT, DH), jnp.float32),
        jax.ShapeDtypeStruct((NS, 1, N_ENT), jnp.float32),
    ],
    mesh=_mesh,
    scratch_types=[
        pltpu.VMEM((C,), jnp.int32),
        pltpu.VMEM((C,), jnp.int32),
        pltpu.VMEM((C,), jnp.float32),
        pltpu.VMEM((C, DH), jnp.float32),
        pltpu.VMEM((1, N_ENT), jnp.float32),
        pltpu.VMEM_SHARED((N_ENT, DH), jnp.float32),
        pltpu.SemaphoreType.DMA,
    ],
    compiler_params=_sc_params,
)
def _phase_b(ent_cat, t_idx, heads, p, zeros_h, agg_out, ssum_out,
             tv, hv, pv, tb, ssl, acc_sh, sem):
    c = lax.axis_index("c")
    s = lax.axis_index("s")

    rb = pl.multiple_of(s * RPA, 8)
    pltpu.sync_copy(zeros_h.at[pl.ds(rb, RPA)], acc_sh.at[pl.ds(rb, RPA)])

    @pl.when(s == NS - 1)
    def _():
        pltpu.sync_copy(zeros_h.at[pl.ds(NS * RPA, RTL)],
                        acc_sh.at[pl.ds(NS * RPA, RTL)])

    z16 = jnp.zeros((L,), jnp.float32)

    @pl.when(c == 0)
    def _():
        def zz(j, _):
            ssl[0, pl.ds(j * L, L)] = z16
            return 0
        lax.fori_loop(0, N_ENT // L, zz, 0)

    plsc.subcore_barrier()

    def chunk(i, _):
        eb = s * EW + i * C
        pltpu.sync_copy(t_idx.at[pl.ds(c * N_EDGE + eb, C)], tv)
        pltpu.sync_copy(heads.at[pl.ds(eb, C)], hv)
        pltpu.sync_copy(p.at[pl.ds(eb, C)], pv)
        pltpu.async_copy(ent_cat.at[tv], tb, sem).wait()

        def group(g, _):
            pvec = pv[pl.ds(g * L, L)]
            for j in range(L):
                ps = pvec[j]
                e = g * L + j
                for k in range(DH // L):
                    tb[e, pl.ds(k * L, L)] = tb[e, pl.ds(k * L, L)] * ps
            return 0

        lax.fori_loop(0, C // L, group, 0)

        @pl.when(c == 0)
        def _():
            zi = jnp.zeros((L,), jnp.int32)
            for j in range(C // L):
                hvec = hv[pl.ds(j * L, L)]
                pvec = pv[pl.ds(j * L, L)]
                plsc.addupdate_scatter(ssl, [zi, hvec], pvec)

        pltpu.sync_copy(tb, acc_sh.at[hv], add=True)
        return 0

    lax.fori_loop(0, NCHUNK, chunk, 0)
    plsc.subcore_barrier()
    ob = pl.multiple_of(c * N_ENT + s * RPA, 8)
    pltpu.sync_copy(acc_sh.at[pl.ds(rb, RPA)], agg_out.at[pl.ds(ob, RPA)])

    @pl.when(s == NS - 1)
    def _():
        pltpu.sync_copy(acc_sh.at[pl.ds(NS * RPA, RTL)],
                        agg_out.at[pl.ds(c * N_ENT + NS * RPA, RTL)])

    @pl.when(c == 0)
    def _():
        pltpu.sync_copy(ssl, ssum_out.at[s])


RB = 400


def _final_body(ent_ref, agg_lo_ref, agg_hi_ref, ssum_ref, w_ref, out_ref):
    ones = jnp.ones((NS, 1), jnp.float32)
    ssum = lax.dot_general(ssum_ref[:, 0, :], ones, (((0,), (0,)), ((), ())),
                           preferred_element_type=jnp.float32)
    rec = 1.0 / (ssum + 1e-10)
    agg = jnp.concatenate(
        [agg_lo_ref[...] * rec, agg_hi_ref[...] * rec], axis=1)
    x = ent_ref[...] + agg
    y = lax.dot_general(x, w_ref[...], (((1,), (1,)), ((), ())),
                        preferred_element_type=jnp.float32)
    out_ref[...] = jnp.where(y >= 0, y, 0.2 * y)


def kernel(entity_emb, heads, rels, tails, rel_table, W):
    heads = heads.astype(jnp.int32)
    tails = tails.astype(jnp.int32)
    rels = rels.astype(jnp.int32)

    ent_cat = jnp.concatenate([entity_emb[:, :DH], entity_emb[:, DH:]], axis=0)
    rel_cat = jnp.concatenate([rel_table[:, :DH], rel_table[:, DH:]], axis=0)
    off2 = (jnp.arange(NC, dtype=jnp.int32) * N_ENT)[:, None]
    h_idx = (heads[None, :] + off2).reshape(-1)
    t_idx = (tails[None, :] + off2).reshape(-1)
    r_idx = (rels[None, :] + (jnp.arange(NC, dtype=jnp.int32) * N_REL)[:, None]
             ).reshape(-1)
    zeros_h = jnp.zeros((N_ENT, DH), jnp.float32)

    scores = _phase_a(ent_cat, rel_cat, h_idx, t_idx, r_idx)

    p = pl.pallas_call(
        _mid_body,
        out_shape=jax.ShapeDtypeStruct((1, N_EDGE), jnp.float32),
    )(scores.reshape(NC, N_EDGE))
    p = p.reshape(N_EDGE)

    agg, ssum_parts = _phase_b(ent_cat, t_idx, heads, p, zeros_h)

    out = pl.pallas_call(
        _final_body,
        grid=(1,),
        in_specs=[
            pl.BlockSpec((N_ENT, D), lambda i: (0, 0)),
            pl.BlockSpec((N_ENT, DH), lambda i: (0, 0)),
            pl.BlockSpec((N_ENT, DH), lambda i: (1, 0)),
            pl.BlockSpec((NS, 1, N_ENT), lambda i: (0, 0, 0)),
            pl.BlockSpec((D, D), lambda i: (0, 0)),
        ],
        out_specs=pl.BlockSpec((N_ENT, D), lambda i: (0, 0)),
        out_shape=jax.ShapeDtypeStruct((N_ENT, D), jnp.float32),
        compiler_params=pltpu.CompilerParams(vmem_limit_bytes=100 << 20),
    )(entity_emb, agg, agg, ssum_parts, W)
    return out


# speedup vs baseline: 3.7970x; 1.7572x over previous
import functools

import jax
import jax.numpy as jnp
from jax import lax
from jax.experimental import pallas as pl
from jax.experimental.pallas import tpu as pltpu
from jax.experimental.pallas import tpu_sc as plsc

N_ENT = 10000
N_REL = 64
N_EDGE = 160000
D = 256
DH = 128
NC = 2
NS = 16
L = 16
EW = N_EDGE // NS
C = 80
NCHUNK = EW // C
NPAIR = (NCHUNK - 1) // 2
RPA = 624
RTL = N_ENT - NS * RPA

_mesh = plsc.VectorSubcoreMesh(core_axis_name="c", subcore_axis_name="s")
_sc_params = pltpu.CompilerParams(needs_layout_passes=False)


def _tanh_via_exp(x):
    ex = jnp.exp(x + x)
    return 1.0 - 2.0 / (ex + 1.0)


@functools.partial(
    pl.kernel,
    out_type=jax.ShapeDtypeStruct((NC * N_EDGE,), jnp.float32),
    mesh=_mesh,
    scratch_types=[
        pltpu.VMEM((NCHUNK, C), jnp.int32),
        pltpu.VMEM((NCHUNK, C), jnp.int32),
        pltpu.VMEM((NCHUNK, C), jnp.int32),
        pltpu.VMEM((C, DH), jnp.float32),
        pltpu.VMEM((C, DH), jnp.float32),
        pltpu.VMEM((C, DH), jnp.float32),
        pltpu.VMEM((C, DH), jnp.float32),
        pltpu.VMEM((C, DH), jnp.float32),
        pltpu.VMEM((C, DH), jnp.float32),
        pltpu.VMEM((EW,), jnp.float32),
        pltpu.SemaphoreType.DMA,
        pltpu.SemaphoreType.DMA,
    ],
    compiler_params=_sc_params,
)
def _phase_a(ent_cat, rel_cat, hx, tx, rx, scores_out,
             hidx, tidx, ridx, eh0, et0, er0, eh1, et1, er1, sball,
             sem0, sem1):
    c = lax.axis_index("c")
    s = lax.axis_index("s")
    w = c * NS + s

    bufs = ((eh0, et0, er0, sem0), (eh1, et1, er1, sem1))

    pltpu.sync_copy(hx.at[w], hidx)
    pltpu.sync_copy(tx.at[w], tidx)
    pltpu.sync_copy(rx.at[w], ridx)

    z16 = jnp.zeros((L,), jnp.float32)

    def zz(j, _):
        sball[pl.ds(j * L, L)] = z16
        return 0

    lax.fori_loop(0, EW // L, zz, 0)

    def issue(i, b):
        eh, et, er, sem = bufs[b]
        pltpu.async_copy(ent_cat.at[hidx.at[i]], eh, sem)
        pltpu.async_copy(ent_cat.at[tidx.at[i]], et, sem)
        pltpu.async_copy(rel_cat.at[ridx.at[i]], er, sem)

    def drain(b):
        eh, et, er, sem = bufs[b]
        pltpu.make_async_copy(ent_cat.at[pl.ds(0, C)], eh, sem).wait()
        pltpu.make_async_copy(ent_cat.at[pl.ds(0, C)], et, sem).wait()
        pltpu.make_async_copy(rel_cat.at[pl.ds(0, C)], er, sem).wait()

    zi16 = jnp.zeros((L,), jnp.int32)

    def compute(i, b):
        eh, et, er, _ = bufs[b]

        def edge(e, _):
            acc = jnp.zeros((L,), jnp.float32)
            for k in range(DH // L):
                vh = eh[e, pl.ds(k * L, L)]
                vr = er[e, pl.ds(k * L, L)]
                vt = et[e, pl.ds(k * L, L)]
                acc = acc + vt * _tanh_via_exp(vh + vr)
            plsc.addupdate_scatter(sball, [zi16 + (i * C + e)], acc)
            return 0

        lax.fori_loop(0, C, edge, 0, unroll=2)

    issue(0, 0)

    def pair(j, _):
        issue(2 * j + 1, 1)
        drain(0)
        compute(2 * j, 0)
        issue(2 * j + 2, 0)
        drain(1)
        compute(2 * j + 1, 1)
        return 0

    lax.fori_loop(0, NPAIR, pair, 0)
    drain(0)
    compute(NCHUNK - 1, 0)

    pltpu.sync_copy(sball, scores_out.at[pl.ds(c * N_EDGE + s * EW, EW)])


def _mid_body(sp_ref, p_ref):
    sc = sp_ref[0, :] + sp_ref[1, :]
    m = jnp.max(sc)
    p_ref[...] = jnp.exp(sc - m)[None, :]


@functools.partial(
    pl.kernel,
    out_type=[
        jax.ShapeDtypeStruct((NC * N_ENT, DH), jnp.float32),
        jax.ShapeDtypeStruct((NS, 1, N_ENT), jnp.float32),
    ],
    mesh=_mesh,
    scratch_types=[
        pltpu.VMEM((C,), jnp.int32),
        pltpu.VMEM((C,), jnp.int32),
        pltpu.VMEM((C,), jnp.int32),
        pltpu.VMEM((C,), jnp.int32),
        pltpu.VMEM((C,), jnp.float32),
        pltpu.VMEM((C,), jnp.float32),
        pltpu.VMEM((C, DH), jnp.float32),
        pltpu.VMEM((C, DH), jnp.float32),
        pltpu.VMEM((1, N_ENT), jnp.float32),
        pltpu.VMEM_SHARED((N_ENT, DH), jnp.float32),
        pltpu.SemaphoreType.DMA,
        pltpu.SemaphoreType.DMA,
        pltpu.SemaphoreType.DMA,
    ],
    compiler_params=_sc_params,
)
def _phase_b(ent_cat, tx, hx, p, zeros_h, agg_out, ssum_out,
             ti0, ti1, hi0, hi1, pb0, pb1, tb0, tb1, ssl, acc_sh,
             sem0, sem1, isem):
    c = lax.axis_index("c")
    s = lax.axis_index("s")
    w = c * NS + s

    bufs = ((ti0, hi0, pb0, tb0, sem0), (ti1, hi1, pb1, tb1, sem1))

    pltpu.sync_copy(tx.at[w, 0], ti0)
    pltpu.sync_copy(hx.at[s, 0], hi0)
    pltpu.sync_copy(p.at[pl.ds(s * EW, C)], pb0)

    rb = pl.multiple_of(s * RPA, 8)
    pltpu.sync_copy(zeros_h.at[pl.ds(rb, RPA)], acc_sh.at[pl.ds(rb, RPA)])

    @pl.when(s == NS - 1)
    def _():
        pltpu.sync_copy(zeros_h.at[pl.ds(NS * RPA, RTL)],
                        acc_sh.at[pl.ds(NS * RPA, RTL)])

    z16 = jnp.zeros((L,), jnp.float32)

    @pl.when(c == 0)
    def _():
        def zz(j, _):
            ssl[0, pl.ds(j * L, L)] = z16
            return 0
        lax.fori_loop(0, N_ENT // L, zz, 0)

    plsc.subcore_barrier()

    def issue(b):
        ti, _, _, tb, sem = bufs[b]
        pltpu.async_copy(ent_cat.at[ti], tb, sem)

    def drain(b):
        _, _, _, tb, sem = bufs[b]
        pltpu.make_async_copy(ent_cat.at[pl.ds(0, C)], tb, sem).wait()

    zi = jnp.zeros((L,), jnp.int32)

    def step(i, b, nxt):
        _, hi, pb, tb, _ = bufs[b]
        nb = 1 - b
        nti, nhi, npb, _, _ = bufs[nb]
        if nxt is not None:
            pltpu.async_copy(tx.at[w, nxt], nti, isem)
            pltpu.async_copy(hx.at[s, nxt], nhi, isem)
            pltpu.async_copy(p.at[pl.ds(s * EW + nxt * C, C)], npb, isem)
        drain(b)

        def group(g, _):
            pvec = pb[pl.ds(g * L, L)]
            for j in range(L):
                ps = pvec[j]
                e = g * L + j
                for k in range(DH // L):
                    tb[e, pl.ds(k * L, L)] = tb[e, pl.ds(k * L, L)] * ps
            return 0

        lax.fori_loop(0, C // L, group, 0)

        @pl.when(c == 0)
        def _():
            for j in range(C // L):
                hvec = hi[pl.ds(j * L, L)]
                pvec = pb[pl.ds(j * L, L)]
                plsc.addupdate_scatter(ssl, [zi, hvec], pvec)

        if nxt is not None:
            pltpu.make_async_copy(tx.at[w, 0], nti, isem).wait()
            pltpu.make_async_copy(hx.at[s, 0], nhi, isem).wait()
            pltpu.make_async_copy(p.at[pl.ds(0, C)], npb, isem).wait()
            issue(nb)
        pltpu.sync_copy(tb, acc_sh.at[hi], add=True)

    issue(0)

    def pair(j, _):
        step(2 * j, 0, 2 * j + 1)
        step(2 * j + 1, 1, 2 * j + 2)
        return 0

    lax.fori_loop(0, NPAIR, pair, 0)
    step(NCHUNK - 1, 0, None)

    plsc.subcore_barrier()
    ob = pl.multiple_of(c * N_ENT + s * RPA, 8)
    pltpu.sync_copy(acc_sh.at[pl.ds(rb, RPA)], agg_out.at[pl.ds(ob, RPA)])

    @pl.when(s == NS - 1)
    def _():
        pltpu.sync_copy(acc_sh.at[pl.ds(NS * RPA, RTL)],
                        agg_out.at[pl.ds(c * N_ENT + NS * RPA, RTL)])

    @pl.when(c == 0)
    def _():
        pltpu.sync_copy(ssl, ssum_out.at[s])


def _final_body(ent_ref, agg_lo_ref, agg_hi_ref, ssum_ref, w_ref, out_ref):
    ones = jnp.ones((NS, 1), jnp.float32)
    ssum = lax.dot_general(ssum_ref[:, 0, :], ones, (((0,), (0,)), ((), ())),
                           preferred_element_type=jnp.float32)
    rec = 1.0 / (ssum + 1e-10)
    agg = jnp.concatenate(
        [agg_lo_ref[...] * rec, agg_hi_ref[...] * rec], axis=1)
    x = ent_ref[...] + agg
    y = lax.dot_general(x, w_ref[...], (((1,), (1,)), ((), ())),
                        preferred_element_type=jnp.float32)
    out_ref[...] = jnp.where(y >= 0, y, 0.2 * y)


def kernel(entity_emb, heads, rels, tails, rel_table, W):
    heads = heads.astype(jnp.int32)
    tails = tails.astype(jnp.int32)
    rels = rels.astype(jnp.int32)

    ent_cat = jnp.concatenate([entity_emb[:, :DH], entity_emb[:, DH:]], axis=0)
    rel_cat = jnp.concatenate([rel_table[:, :DH], rel_table[:, DH:]], axis=0)
    off2 = (jnp.arange(NC, dtype=jnp.int32) * N_ENT)[:, None]
    h_idx = (heads[None, :] + off2).reshape(NC * NS, NCHUNK, C)
    t_idx = (tails[None, :] + off2).reshape(NC * NS, NCHUNK, C)
    r_idx = (rels[None, :] + (jnp.arange(NC, dtype=jnp.int32) * N_REL)[:, None]
             ).reshape(NC * NS, NCHUNK, C)
    heads_w = heads.reshape(NS, NCHUNK, C)
    zeros_h = jnp.zeros((N_ENT, DH), jnp.float32)

    scores = _phase_a(ent_cat, rel_cat, h_idx, t_idx, r_idx)

    p = pl.pallas_call(
        _mid_body,
        out_shape=jax.ShapeDtypeStruct((1, N_EDGE), jnp.float32),
    )(scores.reshape(NC, N_EDGE))
    p = p.reshape(N_EDGE)

    agg, ssum_parts = _phase_b(ent_cat, t_idx, heads_w, p, zeros_h)

    out = pl.pallas_call(
        _final_body,
        grid=(1,),
        in_specs=[
            pl.BlockSpec((N_ENT, D), lambda i: (0, 0)),
            pl.BlockSpec((N_ENT, DH), lambda i: (0, 0)),
            pl.BlockSpec((N_ENT, DH), lambda i: (1, 0)),
            pl.BlockSpec((NS, 1, N_ENT), lambda i: (0, 0, 0)),
            pl.BlockSpec((D, D), lambda i: (0, 0)),
        ],
        out_specs=pl.BlockSpec((N_ENT, D), lambda i: (0, 0)),
        out_shape=jax.ShapeDtypeStruct((N_ENT, D), jnp.float32),
        compiler_params=pltpu.CompilerParams(vmem_limit_bytes=100 << 20),
    )(entity_emb, agg, agg, ssum_parts, W)
    return out
